# Initial kernel scaffold; baseline (speedup 1.0000x reference)
import jax
import jax.numpy as jnp
from jax.experimental import pallas as pl


def kernel(users, items, popularity):
    raise NotImplementedError("write your pallas kernel here")


# speedup vs baseline: 157.6854x; 157.6854x over previous
import functools

import jax
import jax.numpy as jnp
from jax import lax
from jax.experimental import pallas as pl
from jax.experimental.pallas import tpu as pltpu, tpu_sc as plsc

BATCH = 16384
HIST = 200
VOCAB = 1000
N = BATCH * HIST
NW = 32
PER_W = N // NW
CHUNK = 25600
NCHUNK = PER_W // CHUNK
L = 16


def _gather_body(items_hbm, pop_hbm, out_hbm, table_v, idx_v, val_v, sems):
    wid = lax.axis_index("s") * 2 + lax.axis_index("c")
    base = wid * PER_W

    pltpu.sync_copy(pop_hbm, table_v)

    for c in range(NCHUNK):
        off = base + c * CHUNK
        pltpu.sync_copy(items_hbm.at[pl.ds(off, CHUNK)], idx_v)

        def body(i, _):
            sl = pl.ds(i * L, L)
            val_v[sl] = plsc.load_gather(table_v, [idx_v[sl]])
            return ()

        lax.fori_loop(0, CHUNK // L, body, (), unroll=8)
        pltpu.sync_copy(val_v, out_hbm.at[pl.ds(off, CHUNK)])


@functools.partial(jax.jit, static_argnums=())
def _run(items_flat, popularity):
    mesh = plsc.VectorSubcoreMesh(core_axis_name="c", subcore_axis_name="s")
    k = pl.kernel(
        _gather_body,
        out_type=jax.ShapeDtypeStruct((N,), jnp.float32),
        mesh=mesh,
        scratch_types=[
            pltpu.VMEM((VOCAB,), jnp.float32),
            pltpu.VMEM((CHUNK,), jnp.int32),
            pltpu.VMEM((CHUNK,), jnp.float32),
            pltpu.SemaphoreType.DMA,
        ],
        compiler_params=pltpu.CompilerParams(needs_layout_passes=False),
    )
    return k(items_flat, popularity)


def kernel(users, items, popularity):
    del users
    out = _run(items.reshape(-1), popularity)
    return out.reshape(BATCH, HIST)


# speedup vs baseline: 160.1416x; 1.0156x over previous
import functools

import jax
import jax.numpy as jnp
from jax import lax
from jax.experimental import pallas as pl
from jax.experimental.pallas import tpu as pltpu, tpu_sc as plsc

BATCH = 16384
HIST = 200
VOCAB = 1000
N = BATCH * HIST
NW = 32
PER_W = N // NW
CHUNK = 12800
NCHUNK = PER_W // CHUNK
L = 16


def _gather_body(items_hbm, pop_hbm, out_hbm, table_v, idx_v, val_v,
                 in_sems, out_sems):
    wid = lax.axis_index("s") * 2 + lax.axis_index("c")
    base = wid * PER_W

    pltpu.sync_copy(pop_hbm, table_v)

    def in_copy(c, b):
        off = base + c * CHUNK
        return pltpu.make_async_copy(
            items_hbm.at[pl.ds(off, CHUNK)], idx_v.at[b], in_sems[b])

    def out_copy(c, b):
        off = base + c * CHUNK
        return pltpu.make_async_copy(
            val_v.at[b], out_hbm.at[pl.ds(off, CHUNK)], out_sems[b])

    in_copy(0, 0).start()
    for c in range(NCHUNK):
        b = c % 2
        if c + 1 < NCHUNK:
            in_copy(c + 1, 1 - b).start()
        in_copy(c, b).wait()
        if c >= 2:
            out_copy(c - 2, b).wait()

        def body(i, _):
            sl = pl.ds(i * L, L)
            val_v[b, sl] = plsc.load_gather(table_v, [idx_v[b, sl]])
            return ()

        lax.fori_loop(0, CHUNK // L, body, (), unroll=8)

        out_copy(c, b).start()

    out_copy(NCHUNK - 2, NCHUNK % 2).wait()
    out_copy(NCHUNK - 1, (NCHUNK - 1) % 2).wait()


@jax.jit
def _run(items_flat, popularity):
    mesh = plsc.VectorSubcoreMesh(core_axis_name="c", subcore_axis_name="s")
    k = pl.kernel(
        _gather_body,
        out_type=jax.ShapeDtypeStruct((N,), jnp.float32),
        mesh=mesh,
        scratch_types=[
            pltpu.VMEM((VOCAB,), jnp.float32),
            pltpu.VMEM((2, CHUNK), jnp.int32),
            pltpu.VMEM((2, CHUNK), jnp.float32),
            [pltpu.SemaphoreType.DMA, pltpu.SemaphoreType.DMA],
            [pltpu.SemaphoreType.DMA, pltpu.SemaphoreType.DMA],
        ],
        compiler_params=pltpu.CompilerParams(needs_layout_passes=False),
    )
    return k(items_flat, popularity)


def kernel(users, items, popularity):
    del users
    out = _run(items.reshape(-1), popularity)
    return out.reshape(BATCH, HIST)


# speedup vs baseline: 224.9756x; 1.4049x over previous
import functools

import jax
import jax.numpy as jnp
from jax import lax
from jax.experimental import pallas as pl
from jax.experimental.pallas import tpu as pltpu, tpu_sc as plsc

BATCH = 16384
HIST = 200
VOCAB = 1000
N = BATCH * HIST
NW = 32
PER_W = N // NW
CHUNK = 12800
NCHUNK = PER_W // CHUNK
L = 16
K = 16


def _gather_body(items_hbm, pop_hbm, out_hbm, table_v, idx_v, val_v,
                 in_sems, out_sems):
    wid = lax.axis_index("s") * 2 + lax.axis_index("c")
    base = wid * PER_W

    pltpu.sync_copy(pop_hbm, table_v)

    def in_copy(c, b):
        off = base + c * CHUNK
        return pltpu.make_async_copy(
            items_hbm.at[pl.ds(off, CHUNK)], idx_v.at[b], in_sems[b])

    def out_copy(c, b):
        off = base + c * CHUNK
        return pltpu.make_async_copy(
            val_v.at[b], out_hbm.at[pl.ds(off, CHUNK)], out_sems[b])

    in_copy(0, 0).start()
    for c in range(NCHUNK):
        b = c % 2
        if c + 1 < NCHUNK:
            in_copy(c + 1, 1 - b).start()
        in_copy(c, b).wait()
        if c >= 2:
            out_copy(c - 2, b).wait()

        def body(i, _):
            base_i = i * (K * L)
            idxs = [idx_v[b, pl.ds(base_i + j * L, L)] for j in range(K)]
            vals = [plsc.load_gather(table_v, [ix]) for ix in idxs]
            for j in range(K):
                val_v[b, pl.ds(base_i + j * L, L)] = vals[j]
            return ()

        lax.fori_loop(0, CHUNK // (K * L), body, ())

        out_copy(c, b).start()

    out_copy(NCHUNK - 2, NCHUNK % 2).wait()
    out_copy(NCHUNK - 1, (NCHUNK - 1) % 2).wait()


@jax.jit
def _run(items_flat, popularity):
    mesh = plsc.VectorSubcoreMesh(core_axis_name="c", subcore_axis_name="s")
    k = pl.kernel(
        _gather_body,
        out_type=jax.ShapeDtypeStruct((N,), jnp.float32),
        mesh=mesh,
        scratch_types=[
            pltpu.VMEM((VOCAB,), jnp.float32),
            pltpu.VMEM((2, CHUNK), jnp.int32),
            pltpu.VMEM((2, CHUNK), jnp.float32),
            [pltpu.SemaphoreType.DMA, pltpu.SemaphoreType.DMA],
            [pltpu.SemaphoreType.DMA, pltpu.SemaphoreType.DMA],
        ],
        compiler_params=pltpu.CompilerParams(needs_layout_passes=False),
    )
    return k(items_flat, popularity)


def kernel(users, items, popularity):
    del users
    out = _run(items.reshape(-1), popularity)
    return out.reshape(BATCH, HIST)


# speedup vs baseline: 407.8402x; 1.8128x over previous
import functools

import jax
import jax.numpy as jnp
from jax import lax
from jax.experimental import pallas as pl
from jax.experimental.pallas import tpu as pltpu, tpu_sc as plsc

BATCH = 16384
HIST = 200
VOCAB = 1000
NW = 32
ROWS_W = BATCH // NW
R = 64
CHUNK = R * HIST
NCHUNK = ROWS_W // R
L = 16
K = 16


def _gather_body(items_hbm, pop_hbm, out_hbm, table_v, idx_v, val_v,
                 in_sems, out_sems):
    wid = lax.axis_index("s") * 2 + lax.axis_index("c")
    cbase = wid * NCHUNK
    items_c = items_hbm.reshape(NW * NCHUNK, R, HIST)
    out_c = out_hbm.reshape(NW * NCHUNK, R, HIST)

    pltpu.sync_copy(pop_hbm, table_v)

    def in_copy(c, b):
        return pltpu.make_async_copy(
            items_c.at[cbase + c], idx_v.at[b], in_sems[b])

    def out_copy(c, b):
        return pltpu.make_async_copy(
            val_v.at[b], out_c.at[cbase + c], out_sems[b])

    iota = lax.iota(jnp.int32, L)
    NG = (HIST + L - 1) // L
    cols = [iota + j * L for j in range(NG)]
    masks = [cols[j] < HIST for j in range(NG)]

    in_copy(0, 0).start()
    for c in range(NCHUNK):
        b = c % 2
        if c + 1 < NCHUNK:
            in_copy(c + 1, 1 - b).start()
        in_copy(c, b).wait()
        if c >= 2:
            out_copy(c - 2, b).wait()

        idx_b = idx_v.at[b]
        val_b = val_v.at[b]

        def body(r, _):
            row = jnp.full((L,), 0, jnp.int32) + r
            idxs = [plsc.load_gather(idx_b, [row, cols[j]], mask=masks[j])
                    for j in range(NG)]
            vals = [plsc.load_gather(table_v, [ix], mask=masks[j])
                    for j, ix in enumerate(idxs)]
            for j in range(NG):
                plsc.store_scatter(val_b, [row, cols[j]], vals[j],
                                   mask=masks[j])
            return ()

        lax.fori_loop(0, R, body, ())

        out_copy(c, b).start()

    out_copy(NCHUNK - 2, NCHUNK % 2).wait()
    out_copy(NCHUNK - 1, (NCHUNK - 1) % 2).wait()


@jax.jit
def _run(items, popularity):
    mesh = plsc.VectorSubcoreMesh(core_axis_name="c", subcore_axis_name="s")
    k = pl.kernel(
        _gather_body,
        out_type=jax.ShapeDtypeStruct((BATCH, HIST), jnp.float32),
        mesh=mesh,
        scratch_types=[
            pltpu.VMEM((VOCAB,), jnp.float32),
            pltpu.VMEM((2, R, HIST), jnp.int32),
            pltpu.VMEM((2, R, HIST), jnp.float32),
            [pltpu.SemaphoreType.DMA, pltpu.SemaphoreType.DMA],
            [pltpu.SemaphoreType.DMA, pltpu.SemaphoreType.DMA],
        ],
        compiler_params=pltpu.CompilerParams(needs_layout_passes=False),
    )
    return k(items, popularity)


def kernel(users, items, popularity):
    del users
    return _run(items, popularity)


# speedup vs baseline: 408.1163x; 1.0007x over previous
import functools

import jax
import jax.numpy as jnp
from jax import lax
from jax.experimental import pallas as pl
from jax.experimental.pallas import tpu as pltpu, tpu_sc as plsc

BATCH = 16384
HIST = 200
VOCAB = 1000
NW = 32
ROWS_W = BATCH // NW
R = 64
CHUNK = R * HIST
NCHUNK = ROWS_W // R
L = 16
K = 16


def _gather_body(items_hbm, pop_hbm, out_hbm, table_v, idx_v, val_v,
                 in_sems, out_sems):
    wid = lax.axis_index("s") * 2 + lax.axis_index("c")
    cbase = wid * NCHUNK
    items_c = items_hbm.reshape(NW * NCHUNK, R, HIST)
    out_c = out_hbm.reshape(NW * NCHUNK, R, HIST)

    pltpu.sync_copy(pop_hbm, table_v)

    def in_copy(c, b):
        return pltpu.make_async_copy(
            items_c.at[cbase + c], idx_v.at[b], in_sems[b])

    def out_copy(c, b):
        return pltpu.make_async_copy(
            val_v.at[b], out_c.at[cbase + c], out_sems[b])

    iota = lax.iota(jnp.int32, L)
    NG = (HIST + L - 1) // L
    cols = [iota + j * L for j in range(NG)]
    masks = [cols[j] < HIST for j in range(NG)]

    in_copy(0, 0).start()
    for c in range(NCHUNK):
        b = c % 2
        if c + 1 < NCHUNK:
            in_copy(c + 1, 1 - b).start()
        in_copy(c, b).wait()
        if c >= 2:
            out_copy(c - 2, b).wait()

        idx_b = idx_v.at[b]
        val_b = val_v.at[b]

        def body(r, _):
            row = jnp.full((L,), 0, jnp.int32) + r
            idxs = [plsc.load_gather(idx_b, [row, cols[j]], mask=masks[j])
                    for j in range(NG)]
            vals = [plsc.load_gather(table_v, [ix], mask=masks[j])
                    for j, ix in enumerate(idxs)]
            for j in range(NG):
                plsc.store_scatter(val_b, [row, cols[j]], vals[j],
                                   mask=masks[j])
            return ()

        lax.fori_loop(0, R, body, ())

        out_copy(c, b).start()

    out_copy(NCHUNK - 2, NCHUNK % 2).wait()
    out_copy(NCHUNK - 1, (NCHUNK - 1) % 2).wait()


@jax.jit
def _run(items, popularity):
    mesh = plsc.VectorSubcoreMesh(core_axis_name="c", subcore_axis_name="s")
    k = pl.kernel(
        _gather_body,
        out_type=jax.ShapeDtypeStruct((BATCH, HIST), jnp.float32),
        mesh=mesh,
        scratch_types=[
            pltpu.VMEM((VOCAB,), jnp.float32),
            pltpu.VMEM((2, R, HIST), jnp.int32),
            pltpu.VMEM((2, R, HIST), jnp.float32),
            [pltpu.SemaphoreType.DMA, pltpu.SemaphoreType.DMA],
            [pltpu.SemaphoreType.DMA, pltpu.SemaphoreType.DMA],
        ],
        compiler_params=pltpu.CompilerParams(needs_layout_passes=False, use_tc_tiling_on_sc=True),
    )
    return k(items, popularity)


def kernel(users, items, popularity):
    del users
    return _run(items, popularity)
